# Initial kernel scaffold; baseline (speedup 1.0000x reference)
import jax
import jax.numpy as jnp
from jax.experimental import pallas as pl


def kernel(edge_feat, node_feat, state_feat, beW, beB, bnW, bnB, bsW, bsB, ceW0, ceB0, ceW1, ceB1, cnW0, cnB0, cnW1, cnB1, csW0, csB0, csW1, csB1, edge_index):
    raise NotImplementedError("write your pallas kernel here")


# speedup vs baseline: 2.6566x; 2.6566x over previous
import functools

import jax
import jax.numpy as jnp
from jax import lax
from jax.experimental import pallas as pl
from jax.experimental.pallas import tpu as pltpu
from jax.experimental.pallas import tpu_sc as plsc

N = 10000
E = 320000
D = 128
NPAD = 10240
NW = 32
CH = 128
KJ = 80
EW = CH * KJ
EP = NW * EW
TRASH = NPAD - 1

_f32 = jnp.float32


def _p0_body(nf, sf, bnW, bnB, bsW, bsB, Wa, Wb, Wd, ceB0, Wnc, cnB0,
             n_o, A_o, B_o, st_o, crow_e_o, crow_n_o):
    i = pl.program_id(0)
    n = jax.nn.relu(jnp.dot(nf[...], bnW[...], preferred_element_type=_f32)
                    + bnB[...])
    n_o[...] = n
    A_o[...] = jnp.dot(n, Wa[...], preferred_element_type=_f32)
    B_o[...] = jnp.dot(n, Wb[...], preferred_element_type=_f32)

    @pl.when(i == 0)
    def _():
        st = jax.nn.relu(jnp.dot(sf[...], bsW[...],
                                 preferred_element_type=_f32) + bsB[...])
        st_o[...] = st
        crow_e_o[...] = jnp.dot(st, Wd[...],
                                preferred_element_type=_f32) + ceB0[...]
        crow_n_o[...] = jnp.dot(st, Wnc[...],
                                preferred_element_type=_f32) + cnB0[...]


def _phase0(nf_pad, sf, bnW, bnB, bsW, bsB, Wa, Wb, Wd, ceB0, Wnc, cnB0):
    BLK = 1024
    grid = NPAD // BLK
    full = pl.BlockSpec((D, D), lambda i: (0, 0))
    row = pl.BlockSpec((1, D), lambda i: (0, 0))
    blk = pl.BlockSpec((BLK, D), lambda i: (i, 0))
    return pl.pallas_call(
        _p0_body,
        grid=(grid,),
        in_specs=[blk, row, full, row, full, row, full, full, full, row,
                  full, row],
        out_specs=[blk, blk, blk, row, row, row],
        out_shape=[
            jax.ShapeDtypeStruct((NPAD, D), _f32),
            jax.ShapeDtypeStruct((NPAD, D), _f32),
            jax.ShapeDtypeStruct((NPAD, D), _f32),
            jax.ShapeDtypeStruct((1, D), _f32),
            jax.ShapeDtypeStruct((1, D), _f32),
            jax.ShapeDtypeStruct((1, D), _f32),
        ],
        compiler_params=pltpu.CompilerParams(
            dimension_semantics=("arbitrary",)),
    )(nf_pad, sf, bnW, bnB.reshape(1, D), bsW, bsB.reshape(1, D), Wa, Wb,
      Wd, ceB0.reshape(1, D), Wnc, cnB0.reshape(1, D))


def _p1_body(A_hbm, B_hbm, srcT, dstT, GA_hbm, GB_hbm,
             src_v, dst_v, bufA, bufB, semA, semB):
    c = lax.axis_index("c")
    s = lax.axis_index("s")
    wid = s * 2 + c
    base = wid * KJ
    pltpu.sync_copy(srcT.at[pl.ds(base, KJ)], src_v)
    pltpu.sync_copy(dstT.at[pl.ds(base, KJ)], dst_v)

    def body(j, carry):
        r0 = (base + j) * CH
        cpA = pltpu.async_copy(A_hbm.at[src_v.at[j]], bufA, semA)
        cpB = pltpu.async_copy(B_hbm.at[dst_v.at[j]], bufB, semB)
        cpA.wait()
        cpB.wait()
        pltpu.sync_copy(bufA, GA_hbm.at[pl.ds(r0, CH)])
        pltpu.sync_copy(bufB, GB_hbm.at[pl.ds(r0, CH)])
        return carry

    lax.fori_loop(0, KJ, body, 0)


def _phase1(A, B, srcT, dstT):
    mesh = plsc.VectorSubcoreMesh(core_axis_name="c", subcore_axis_name="s")
    f = functools.partial(
        pl.kernel,
        mesh=mesh,
        out_type=[
            jax.ShapeDtypeStruct((EP, D), _f32),
            jax.ShapeDtypeStruct((EP, D), _f32),
        ],
        scratch_types=[
            pltpu.VMEM((KJ, CH), jnp.int32),
            pltpu.VMEM((KJ, CH), jnp.int32),
            pltpu.VMEM((CH, D), _f32),
            pltpu.VMEM((CH, D), _f32),
            pltpu.SemaphoreType.DMA,
            pltpu.SemaphoreType.DMA,
        ],
    )(_p1_body)
    return f(A, B, srcT, dstT)


def _p2_body(ef, ga, gb, beW, beB, Wc, crow, ceW1, ceB1,
             eo_o, eu_o, esum_o):
    i = pl.program_id(0)
    e = jax.nn.relu(jnp.dot(ef[...], beW[...], preferred_element_type=_f32)
                    + beB[...])
    h = jax.nn.relu(jnp.dot(e, Wc[...], preferred_element_type=_f32)
                    + ga[...] + gb[...] + crow[...])
    eu = jax.nn.relu(jnp.dot(h, ceW1[...], preferred_element_type=_f32)
                     + ceB1[...])
    eu_o[...] = eu
    eo_o[...] = eu + ef[...]
    part = jnp.sum(eu, axis=0, keepdims=True)

    @pl.when(i == 0)
    def _():
        esum_o[...] = part

    @pl.when(i > 0)
    def _():
        esum_o[...] = esum_o[...] + part


def _phase2(ef, GA, GB, beW, beB, Wc, crow_e, ceW1, ceB1):
    BLK = 3200
    grid = E // BLK
    full = pl.BlockSpec((D, D), lambda i: (0, 0))
    row = pl.BlockSpec((1, D), lambda i: (0, 0))
    blk = pl.BlockSpec((BLK, D), lambda i: (i, 0))
    return pl.pallas_call(
        _p2_body,
        grid=(grid,),
        in_specs=[blk, blk, blk, full, row, full, row, full, row],
        out_specs=[blk, blk, row],
        out_shape=[
            jax.ShapeDtypeStruct((E, D), _f32),
            jax.ShapeDtypeStruct((EP, D), _f32),
            jax.ShapeDtypeStruct((1, D), _f32),
        ],
        compiler_params=pltpu.CompilerParams(
            dimension_semantics=("arbitrary",)),
    )(ef, GA, GB, beW, beB.reshape(1, D), Wc, crow_e, ceW1,
      ceB1.reshape(1, D))


def _p3_body(eu_hbm, dstT, sums_hbm, counts_hbm,
             dst_v, buf, ones1, zb1, acc_sh, cnt_sh):
    c = lax.axis_index("c")
    s = lax.axis_index("s")
    wid = s * 2 + c
    base = wid * KJ
    pltpu.sync_copy(dstT.at[pl.ds(base, KJ)], dst_v)

    z16 = jnp.zeros((16,), _f32)
    ones = jnp.full((16,), 1.0, _f32)

    def zero_row(i, carry):
        for t in range(8):
            buf[i, pl.ds(t * 16, 16)] = z16
        return carry

    lax.fori_loop(0, CH, zero_row, 0)

    def fill_ones(i, carry):
        ones1[pl.ds(i * 16, 16)] = ones
        return carry

    lax.fori_loop(0, CH // 16, fill_ones, 0)

    def zero_z(i, carry):
        zb1[pl.ds(i * 16, 16)] = z16
        return carry

    lax.fori_loop(0, 640 // 16, zero_z, 0)

    for k in range(5):
        pltpu.sync_copy(buf, acc_sh.at[pl.ds(s * 640 + k * CH, CH)])
    pltpu.sync_copy(zb1, cnt_sh.at[pl.ds(s * 640, 640)])

    plsc.subcore_barrier()

    def body(j, carry):
        r0 = (base + j) * CH
        pltpu.sync_copy(eu_hbm.at[pl.ds(r0, CH)], buf)
        pltpu.sync_copy(buf, acc_sh.at[dst_v.at[j]], add=True)
        pltpu.sync_copy(ones1, cnt_sh.at[dst_v.at[j]], add=True)
        return carry

    lax.fori_loop(0, KJ, body, 0)

    plsc.subcore_barrier()

    pltpu.sync_copy(cnt_sh.at[pl.ds(s * 640, 640)], zb1)
    pltpu.sync_copy(zb1, counts_hbm.at[c, pl.ds(s * 640, 640)])
    for k in range(5):
        r = s * 640 + k * CH
        pltpu.sync_copy(acc_sh.at[pl.ds(r, CH)], buf)
        pltpu.sync_copy(buf, sums_hbm.at[c, pl.ds(r, CH)])


def _phase3(eu_pad, dstT):
    mesh = plsc.VectorSubcoreMesh(core_axis_name="c", subcore_axis_name="s")
    f = functools.partial(
        pl.kernel,
        mesh=mesh,
        out_type=[
            jax.ShapeDtypeStruct((2, NPAD, D), _f32),
            jax.ShapeDtypeStruct((2, NPAD), _f32),
        ],
        scratch_types=[
            pltpu.VMEM((KJ, CH), jnp.int32),
            pltpu.VMEM((CH, D), _f32),
            pltpu.VMEM((CH,), _f32),
            pltpu.VMEM((640,), _f32),
            pltpu.VMEM_SHARED((NPAD, D), _f32),
            pltpu.VMEM_SHARED((NPAD,), _f32),
        ],
    )(_p3_body)
    return f(eu_pad, dstT)


def _p4_body(p, cntb, n, nf, crow_n, Wna, Wnb, cnW1, cnB1,
             st, esum, Wsa, Wsb, Wsc, csB0, csW1, csB1, sf,
             no_o, so_o, acc):
    i = pl.program_id(0)
    ng = pl.num_programs(0)
    sums = p[0] + p[1]
    ve = sums / jnp.maximum(cntb[...], 1.0)
    pre = (jnp.dot(n[...], Wna[...], preferred_element_type=_f32)
           + jnp.dot(ve, Wnb[...], preferred_element_type=_f32)
           + crow_n[...])
    nu = jax.nn.relu(jnp.dot(jax.nn.relu(pre), cnW1[...],
                             preferred_element_type=_f32) + cnB1[...])
    no_o[...] = nu + nf[...]

    rid = i * n.shape[0] + lax.broadcasted_iota(jnp.int32, nu.shape, 0)
    num = jnp.where(rid < N, nu, 0.0)
    part = jnp.sum(num, axis=0, keepdims=True)
    tot = jnp.where(i == 0, part, acc[...] + part)
    acc[...] = tot

    @pl.when(i == ng - 1)
    def _():
        u_e = esum[...] * (1.0 / E)
        u_v = tot * (1.0 / N)
        t = jax.nn.relu(
            jnp.dot(st[...], Wsa[...], preferred_element_type=_f32)
            + jnp.dot(u_e, Wsb[...], preferred_element_type=_f32)
            + jnp.dot(u_v, Wsc[...], preferred_element_type=_f32)
            + csB0[...])
        su = jax.nn.relu(jnp.dot(t, csW1[...],
                                 preferred_element_type=_f32) + csB1[...])
        so_o[...] = su + sf[...]


def _phase4(parts, cntb, n_pad, nf_pad, crow_n, Wna, Wnb, cnW1, cnB1,
            st, esum, Wsa, Wsb, Wsc, csB0, csW1, csB1, sf):
    BLK = 1024
    grid = NPAD // BLK
    full = pl.BlockSpec((D, D), lambda i: (0, 0))
    row = pl.BlockSpec((1, D), lambda i: (0, 0))
    blk = pl.BlockSpec((BLK, D), lambda i: (i, 0))
    pblk = pl.BlockSpec((2, BLK, D), lambda i: (0, i, 0))
    return pl.pallas_call(
        _p4_body,
        grid=(grid,),
        in_specs=[pblk, blk, blk, blk, row, full, full, full, row,
                  row, row, full, full, full, row, full, row, row],
        out_specs=[blk, row],
        out_shape=[
            jax.ShapeDtypeStruct((NPAD, D), _f32),
            jax.ShapeDtypeStruct((1, D), _f32),
        ],
        scratch_shapes=[pltpu.VMEM((1, D), _f32)],
        compiler_params=pltpu.CompilerParams(
            dimension_semantics=("arbitrary",)),
    )(parts, cntb, n_pad, nf_pad, crow_n, Wna, Wnb, cnW1,
      cnB1.reshape(1, D), st, esum, Wsa, Wsb, Wsc, csB0.reshape(1, D),
      csW1, csB1.reshape(1, D), sf)


def kernel(edge_feat, node_feat, state_feat, beW, beB, bnW, bnB, bsW, bsB,
           ceW0, ceB0, ceW1, ceB1, cnW0, cnB0, cnW1, cnB1,
           csW0, csB0, csW1, csB1, edge_index):
    Wa = ceW0[0 * D:1 * D]
    Wb = ceW0[1 * D:2 * D]
    Wc = ceW0[2 * D:3 * D]
    Wd = ceW0[3 * D:4 * D]
    Wna = cnW0[0 * D:1 * D]
    Wnb = cnW0[1 * D:2 * D]
    Wnc = cnW0[2 * D:3 * D]
    Wsa = csW0[0 * D:1 * D]
    Wsb = csW0[1 * D:2 * D]
    Wsc = csW0[2 * D:3 * D]

    nf_pad = jnp.pad(node_feat, ((0, NPAD - N), (0, 0)))
    src = edge_index[0]
    dst = edge_index[1]
    srcT = jnp.pad(src, (0, EP - E)).reshape(EP // CH, CH)
    dstT = jnp.pad(dst, (0, EP - E),
                   constant_values=TRASH).reshape(EP // CH, CH)
    n_pad, A, B, st, crow_e, crow_n = _phase0(
        nf_pad, state_feat, bnW, bnB, bsW, bsB, Wa, Wb, Wd, ceB0, Wnc, cnB0)

    GA, GB = _phase1(A, B, srcT, dstT)

    e_out, eu_pad, esum = _phase2(edge_feat, GA, GB, beW, beB, Wc, crow_e,
                                  ceW1, ceB1)

    sums_parts, counts_parts = _phase3(eu_pad, dstT)

    cnt = (counts_parts[0] + counts_parts[1]).reshape(NPAD, 1)
    cntb = jnp.broadcast_to(cnt, (NPAD, D))

    n_out_pad, s_out = _phase4(
        sums_parts, cntb, n_pad, nf_pad, crow_n, Wna, Wnb, cnW1, cnB1,
        st, esum, Wsa, Wsb, Wsc, csB0, csW1, csB1, state_feat)

    return (e_out, n_out_pad[:N], s_out)


# speedup vs baseline: 2.8458x; 1.0712x over previous
import functools

import jax
import jax.numpy as jnp
from jax import lax
from jax.experimental import pallas as pl
from jax.experimental.pallas import tpu as pltpu
from jax.experimental.pallas import tpu_sc as plsc

N = 10000
E = 320000
D = 128
NPAD = 10240
NW = 32
CH = 128
KJ = 80
EW = CH * KJ
EP = NW * EW
TRASH = NPAD - 1

_f32 = jnp.float32


def _p0_body(nf, sf, bnW, bnB, bsW, bsB, Wa, Wb, Wd, ceB0, Wnc, cnB0,
             n_o, A_o, B_o, st_o, crow_e_o, crow_n_o):
    i = pl.program_id(0)
    n = jax.nn.relu(jnp.dot(nf[...], bnW[...], preferred_element_type=_f32)
                    + bnB[...])
    n_o[...] = n
    A_o[...] = jnp.dot(n, Wa[...], preferred_element_type=_f32)
    B_o[...] = jnp.dot(n, Wb[...], preferred_element_type=_f32)

    @pl.when(i == 0)
    def _():
        st = jax.nn.relu(jnp.dot(sf[...], bsW[...],
                                 preferred_element_type=_f32) + bsB[...])
        st_o[...] = st
        crow_e_o[...] = jnp.dot(st, Wd[...],
                                preferred_element_type=_f32) + ceB0[...]
        crow_n_o[...] = jnp.dot(st, Wnc[...],
                                preferred_element_type=_f32) + cnB0[...]


def _phase0(nf_pad, sf, bnW, bnB, bsW, bsB, Wa, Wb, Wd, ceB0, Wnc, cnB0):
    BLK = 1024
    grid = NPAD // BLK
    full = pl.BlockSpec((D, D), lambda i: (0, 0))
    row = pl.BlockSpec((1, D), lambda i: (0, 0))
    blk = pl.BlockSpec((BLK, D), lambda i: (i, 0))
    return pl.pallas_call(
        _p0_body,
        grid=(grid,),
        in_specs=[blk, row, full, row, full, row, full, full, full, row,
                  full, row],
        out_specs=[blk, blk, blk, row, row, row],
        out_shape=[
            jax.ShapeDtypeStruct((NPAD, D), _f32),
            jax.ShapeDtypeStruct((NPAD, D), _f32),
            jax.ShapeDtypeStruct((NPAD, D), _f32),
            jax.ShapeDtypeStruct((1, D), _f32),
            jax.ShapeDtypeStruct((1, D), _f32),
            jax.ShapeDtypeStruct((1, D), _f32),
        ],
        compiler_params=pltpu.CompilerParams(
            dimension_semantics=("arbitrary",)),
    )(nf_pad, sf, bnW, bnB.reshape(1, D), bsW, bsB.reshape(1, D), Wa, Wb,
      Wd, ceB0.reshape(1, D), Wnc, cnB0.reshape(1, D))


def _p1_body(A_hbm, B_hbm, srcT, dstT, GA_hbm, GB_hbm,
             src_v, dst_v, bufA0, bufB0, bufA1, bufB1,
             sgA0, sgB0, sgA1, sgB1, swA0, swB0, swA1, swB1):
    c = lax.axis_index("c")
    s = lax.axis_index("s")
    wid = s * 2 + c
    base = wid * KJ
    pltpu.sync_copy(srcT.at[pl.ds(base, KJ)], src_v)
    pltpu.sync_copy(dstT.at[pl.ds(base, KJ)], dst_v)

    def start_gather(j, bufA, bufB, sA, sB):
        pltpu.async_copy(A_hbm.at[src_v.at[j]], bufA, sA)
        pltpu.async_copy(B_hbm.at[dst_v.at[j]], bufB, sB)

    def wait_gather(bufA, bufB, sA, sB):
        pltpu.make_async_copy(A_hbm.at[src_v.at[0]], bufA, sA).wait()
        pltpu.make_async_copy(B_hbm.at[dst_v.at[0]], bufB, sB).wait()

    def start_write(j, bufA, bufB, sA, sB):
        r0 = (base + j) * CH
        pltpu.async_copy(bufA, GA_hbm.at[pl.ds(r0, CH)], sA)
        pltpu.async_copy(bufB, GB_hbm.at[pl.ds(r0, CH)], sB)

    def wait_write(bufA, bufB, sA, sB):
        pltpu.make_async_copy(bufA, GA_hbm.at[pl.ds(0, CH)], sA).wait()
        pltpu.make_async_copy(bufB, GB_hbm.at[pl.ds(0, CH)], sB).wait()

    start_gather(0, bufA0, bufB0, sgA0, sgB0)
    start_gather(1, bufA1, bufB1, sgA1, sgB1)

    def body(k, carry):
        j0 = 2 * k
        wait_gather(bufA0, bufB0, sgA0, sgB0)
        start_write(j0, bufA0, bufB0, swA0, swB0)
        wait_gather(bufA1, bufB1, sgA1, sgB1)
        start_write(j0 + 1, bufA1, bufB1, swA1, swB1)
        wait_write(bufA0, bufB0, swA0, swB0)
        start_gather(j0 + 2, bufA0, bufB0, sgA0, sgB0)
        wait_write(bufA1, bufB1, swA1, swB1)
        start_gather(j0 + 3, bufA1, bufB1, sgA1, sgB1)
        return carry

    lax.fori_loop(0, KJ // 2 - 1, body, 0)

    wait_gather(bufA0, bufB0, sgA0, sgB0)
    start_write(KJ - 2, bufA0, bufB0, swA0, swB0)
    wait_gather(bufA1, bufB1, sgA1, sgB1)
    start_write(KJ - 1, bufA1, bufB1, swA1, swB1)
    wait_write(bufA0, bufB0, swA0, swB0)
    wait_write(bufA1, bufB1, swA1, swB1)


def _phase1(A, B, srcT, dstT):
    mesh = plsc.VectorSubcoreMesh(core_axis_name="c", subcore_axis_name="s")
    f = functools.partial(
        pl.kernel,
        mesh=mesh,
        out_type=[
            jax.ShapeDtypeStruct((EP, D), _f32),
            jax.ShapeDtypeStruct((EP, D), _f32),
        ],
        scratch_types=[
            pltpu.VMEM((KJ, CH), jnp.int32),
            pltpu.VMEM((KJ, CH), jnp.int32),
            pltpu.VMEM((CH, D), _f32),
            pltpu.VMEM((CH, D), _f32),
            pltpu.VMEM((CH, D), _f32),
            pltpu.VMEM((CH, D), _f32),
        ] + [pltpu.SemaphoreType.DMA] * 8,
    )(_p1_body)
    return f(A, B, srcT, dstT)


def _p2_body(ef, ga, gb, beW, beB, Wc, crow, ceW1, ceB1,
             eo_o, eu_o, esum_o):
    i = pl.program_id(0)
    e = jax.nn.relu(jnp.dot(ef[...], beW[...], preferred_element_type=_f32)
                    + beB[...])
    h = jax.nn.relu(jnp.dot(e, Wc[...], preferred_element_type=_f32)
                    + ga[...] + gb[...] + crow[...])
    eu = jax.nn.relu(jnp.dot(h, ceW1[...], preferred_element_type=_f32)
                     + ceB1[...])
    eu_o[...] = eu
    eo_o[...] = eu + ef[...]
    part = jnp.sum(eu, axis=0, keepdims=True)

    @pl.when(i == 0)
    def _():
        esum_o[...] = part

    @pl.when(i > 0)
    def _():
        esum_o[...] = esum_o[...] + part


def _phase2(ef, GA, GB, beW, beB, Wc, crow_e, ceW1, ceB1):
    BLK = 3200
    grid = E // BLK
    full = pl.BlockSpec((D, D), lambda i: (0, 0))
    row = pl.BlockSpec((1, D), lambda i: (0, 0))
    blk = pl.BlockSpec((BLK, D), lambda i: (i, 0))
    return pl.pallas_call(
        _p2_body,
        grid=(grid,),
        in_specs=[blk, blk, blk, full, row, full, row, full, row],
        out_specs=[blk, blk, row],
        out_shape=[
            jax.ShapeDtypeStruct((E, D), _f32),
            jax.ShapeDtypeStruct((EP, D), _f32),
            jax.ShapeDtypeStruct((1, D), _f32),
        ],
        compiler_params=pltpu.CompilerParams(
            dimension_semantics=("arbitrary",)),
    )(ef, GA, GB, beW, beB.reshape(1, D), Wc, crow_e, ceW1,
      ceB1.reshape(1, D))


def _p3_body(eu_hbm, dstT, sums_hbm, counts_hbm,
             dst_v, buf, ones1, zb1, acc_sh, cnt_sh):
    c = lax.axis_index("c")
    s = lax.axis_index("s")
    wid = s * 2 + c
    base = wid * KJ
    pltpu.sync_copy(dstT.at[pl.ds(base, KJ)], dst_v)

    z16 = jnp.zeros((16,), _f32)
    ones = jnp.full((16,), 1.0, _f32)

    def zero_row(i, carry):
        for t in range(8):
            buf[i, pl.ds(t * 16, 16)] = z16
        return carry

    lax.fori_loop(0, CH, zero_row, 0)

    def fill_ones(i, carry):
        ones1[pl.ds(i * 16, 16)] = ones
        return carry

    lax.fori_loop(0, CH // 16, fill_ones, 0)

    def zero_z(i, carry):
        zb1[pl.ds(i * 16, 16)] = z16
        return carry

    lax.fori_loop(0, 640 // 16, zero_z, 0)

    for k in range(5):
        pltpu.sync_copy(buf, acc_sh.at[pl.ds(s * 640 + k * CH, CH)])
    pltpu.sync_copy(zb1, cnt_sh.at[pl.ds(s * 640, 640)])

    plsc.subcore_barrier()

    def body(j, carry):
        r0 = (base + j) * CH
        pltpu.sync_copy(eu_hbm.at[pl.ds(r0, CH)], buf)
        pltpu.sync_copy(buf, acc_sh.at[dst_v.at[j]], add=True)
        pltpu.sync_copy(ones1, cnt_sh.at[dst_v.at[j]], add=True)
        return carry

    lax.fori_loop(0, KJ, body, 0)

    plsc.subcore_barrier()

    pltpu.sync_copy(cnt_sh.at[pl.ds(s * 640, 640)], zb1)
    pltpu.sync_copy(zb1, counts_hbm.at[c, pl.ds(s * 640, 640)])
    for k in range(5):
        r = s * 640 + k * CH
        pltpu.sync_copy(acc_sh.at[pl.ds(r, CH)], buf)
        pltpu.sync_copy(buf, sums_hbm.at[c, pl.ds(r, CH)])


def _phase3(eu_pad, dstT):
    mesh = plsc.VectorSubcoreMesh(core_axis_name="c", subcore_axis_name="s")
    f = functools.partial(
        pl.kernel,
        mesh=mesh,
        out_type=[
            jax.ShapeDtypeStruct((2, NPAD, D), _f32),
            jax.ShapeDtypeStruct((2, NPAD), _f32),
        ],
        scratch_types=[
            pltpu.VMEM((KJ, CH), jnp.int32),
            pltpu.VMEM((CH, D), _f32),
            pltpu.VMEM((CH,), _f32),
            pltpu.VMEM((640,), _f32),
            pltpu.VMEM_SHARED((NPAD, D), _f32),
            pltpu.VMEM_SHARED((NPAD,), _f32),
        ],
    )(_p3_body)
    return f(eu_pad, dstT)


def _p4_body(p, cntb, n, nf, crow_n, Wna, Wnb, cnW1, cnB1,
             st, esum, Wsa, Wsb, Wsc, csB0, csW1, csB1, sf,
             no_o, so_o, acc):
    i = pl.program_id(0)
    ng = pl.num_programs(0)
    sums = p[0] + p[1]
    ve = sums / jnp.maximum(cntb[...], 1.0)
    pre = (jnp.dot(n[...], Wna[...], preferred_element_type=_f32)
           + jnp.dot(ve, Wnb[...], preferred_element_type=_f32)
           + crow_n[...])
    nu = jax.nn.relu(jnp.dot(jax.nn.relu(pre), cnW1[...],
                             preferred_element_type=_f32) + cnB1[...])
    no_o[...] = nu + nf[...]

    rid = i * n.shape[0] + lax.broadcasted_iota(jnp.int32, nu.shape, 0)
    num = jnp.where(rid < N, nu, 0.0)
    part = jnp.sum(num, axis=0, keepdims=True)
    tot = jnp.where(i == 0, part, acc[...] + part)
    acc[...] = tot

    @pl.when(i == ng - 1)
    def _():
        u_e = esum[...] * (1.0 / E)
        u_v = tot * (1.0 / N)
        t = jax.nn.relu(
            jnp.dot(st[...], Wsa[...], preferred_element_type=_f32)
            + jnp.dot(u_e, Wsb[...], preferred_element_type=_f32)
            + jnp.dot(u_v, Wsc[...], preferred_element_type=_f32)
            + csB0[...])
        su = jax.nn.relu(jnp.dot(t, csW1[...],
                                 preferred_element_type=_f32) + csB1[...])
        so_o[...] = su + sf[...]


def _phase4(parts, cntb, n_pad, nf_pad, crow_n, Wna, Wnb, cnW1, cnB1,
            st, esum, Wsa, Wsb, Wsc, csB0, csW1, csB1, sf):
    BLK = 1024
    grid = NPAD // BLK
    full = pl.BlockSpec((D, D), lambda i: (0, 0))
    row = pl.BlockSpec((1, D), lambda i: (0, 0))
    blk = pl.BlockSpec((BLK, D), lambda i: (i, 0))
    pblk = pl.BlockSpec((2, BLK, D), lambda i: (0, i, 0))
    return pl.pallas_call(
        _p4_body,
        grid=(grid,),
        in_specs=[pblk, blk, blk, blk, row, full, full, full, row,
                  row, row, full, full, full, row, full, row, row],
        out_specs=[blk, row],
        out_shape=[
            jax.ShapeDtypeStruct((NPAD, D), _f32),
            jax.ShapeDtypeStruct((1, D), _f32),
        ],
        scratch_shapes=[pltpu.VMEM((1, D), _f32)],
        compiler_params=pltpu.CompilerParams(
            dimension_semantics=("arbitrary",)),
    )(parts, cntb, n_pad, nf_pad, crow_n, Wna, Wnb, cnW1,
      cnB1.reshape(1, D), st, esum, Wsa, Wsb, Wsc, csB0.reshape(1, D),
      csW1, csB1.reshape(1, D), sf)


def kernel(edge_feat, node_feat, state_feat, beW, beB, bnW, bnB, bsW, bsB,
           ceW0, ceB0, ceW1, ceB1, cnW0, cnB0, cnW1, cnB1,
           csW0, csB0, csW1, csB1, edge_index):
    Wa = ceW0[0 * D:1 * D]
    Wb = ceW0[1 * D:2 * D]
    Wc = ceW0[2 * D:3 * D]
    Wd = ceW0[3 * D:4 * D]
    Wna = cnW0[0 * D:1 * D]
    Wnb = cnW0[1 * D:2 * D]
    Wnc = cnW0[2 * D:3 * D]
    Wsa = csW0[0 * D:1 * D]
    Wsb = csW0[1 * D:2 * D]
    Wsc = csW0[2 * D:3 * D]

    nf_pad = jnp.pad(node_feat, ((0, NPAD - N), (0, 0)))
    src = edge_index[0]
    dst = edge_index[1]
    srcT = jnp.pad(src, (0, EP - E)).reshape(EP // CH, CH)
    dstT = jnp.pad(dst, (0, EP - E),
                   constant_values=TRASH).reshape(EP // CH, CH)
    n_pad, A, B, st, crow_e, crow_n = _phase0(
        nf_pad, state_feat, bnW, bnB, bsW, bsB, Wa, Wb, Wd, ceB0, Wnc, cnB0)

    GA, GB = _phase1(A, B, srcT, dstT)

    e_out, eu_pad, esum = _phase2(edge_feat, GA, GB, beW, beB, Wc, crow_e,
                                  ceW1, ceB1)

    sums_parts, counts_parts = _phase3(eu_pad, dstT)

    cnt = (counts_parts[0] + counts_parts[1]).reshape(NPAD, 1)
    cntb = jnp.broadcast_to(cnt, (NPAD, D))

    n_out_pad, s_out = _phase4(
        sums_parts, cntb, n_pad, nf_pad, crow_n, Wna, Wnb, cnW1, cnB1,
        st, esum, Wsa, Wsb, Wsc, csB0, csW1, csB1, state_feat)

    return (e_out, n_out_pad[:N], s_out)


# speedup vs baseline: 3.0366x; 1.0670x over previous
import functools

import jax
import jax.numpy as jnp
from jax import lax
from jax.experimental import pallas as pl
from jax.experimental.pallas import tpu as pltpu
from jax.experimental.pallas import tpu_sc as plsc

N = 10000
E = 320000
D = 128
NPAD = 10240
NW = 32
CH = 128
KJ = 80
EW = CH * KJ
EP = NW * EW
TRASH = NPAD - 1

_f32 = jnp.float32


def _p0_body(nf, sf, bnW, bnB, bsW, bsB, Wa, Wb, Wd, ceB0, Wnc, cnB0,
             n_o, A_o, B_o, st_o, crow_e_o, crow_n_o):
    i = pl.program_id(0)
    n = jax.nn.relu(jnp.dot(nf[...], bnW[...], preferred_element_type=_f32)
                    + bnB[...])
    n_o[...] = n
    a = jnp.dot(n, Wa[...], preferred_element_type=_f32)
    b = jnp.dot(n, Wb[...], preferred_element_type=_f32)
    A_o[...] = jnp.broadcast_to(a[None], A_o.shape)
    B_o[...] = jnp.broadcast_to(b[None], B_o.shape)

    @pl.when(i == 0)
    def _():
        st = jax.nn.relu(jnp.dot(sf[...], bsW[...],
                                 preferred_element_type=_f32) + bsB[...])
        st_o[...] = st
        crow_e_o[...] = jnp.dot(st, Wd[...],
                                preferred_element_type=_f32) + ceB0[...]
        crow_n_o[...] = jnp.dot(st, Wnc[...],
                                preferred_element_type=_f32) + cnB0[...]


def _phase0(nf_pad, sf, bnW, bnB, bsW, bsB, Wa, Wb, Wd, ceB0, Wnc, cnB0):
    BLK = 1024
    grid = NPAD // BLK
    full = pl.BlockSpec((D, D), lambda i: (0, 0))
    row = pl.BlockSpec((1, D), lambda i: (0, 0))
    blk = pl.BlockSpec((BLK, D), lambda i: (i, 0))
    blk2 = pl.BlockSpec((2, BLK, D), lambda i: (0, i, 0))
    return pl.pallas_call(
        _p0_body,
        grid=(grid,),
        in_specs=[blk, row, full, row, full, row, full, full, full, row,
                  full, row],
        out_specs=[blk, blk2, blk2, row, row, row],
        out_shape=[
            jax.ShapeDtypeStruct((NPAD, D), _f32),
            jax.ShapeDtypeStruct((2, NPAD, D), _f32),
            jax.ShapeDtypeStruct((2, NPAD, D), _f32),
            jax.ShapeDtypeStruct((1, D), _f32),
            jax.ShapeDtypeStruct((1, D), _f32),
            jax.ShapeDtypeStruct((1, D), _f32),
        ],
        compiler_params=pltpu.CompilerParams(
            dimension_semantics=("arbitrary",)),
    )(nf_pad, sf, bnW, bnB.reshape(1, D), bsW, bsB.reshape(1, D), Wa, Wb,
      Wd, ceB0.reshape(1, D), Wnc, cnB0.reshape(1, D))


def _p1_body(A_hbm, B_hbm, srcT, dstT, GA_hbm, GB_hbm,
             src_v, dst_v, bufA0, bufB0, bufA1, bufB1,
             sgA0, sgB0, sgA1, sgB1, swA0, swB0, swA1, swB1):
    c = lax.axis_index("c")
    s = lax.axis_index("s")
    wid = s * 2 + c
    base = wid * KJ
    pltpu.sync_copy(srcT.at[pl.ds(base, KJ)], src_v)
    pltpu.sync_copy(dstT.at[pl.ds(base, KJ)], dst_v)

    def start_gather(j, bufA, bufB, sA, sB):
        pltpu.async_copy(A_hbm.at[c].at[src_v.at[j]], bufA, sA)
        pltpu.async_copy(B_hbm.at[c].at[dst_v.at[j]], bufB, sB)

    def wait_gather(bufA, bufB, sA, sB):
        pltpu.make_async_copy(A_hbm.at[c].at[src_v.at[0]], bufA, sA).wait()
        pltpu.make_async_copy(B_hbm.at[c].at[dst_v.at[0]], bufB, sB).wait()

    def start_write(j, bufA, bufB, sA, sB):
        r0 = (base + j) * CH
        pltpu.async_copy(bufA, GA_hbm.at[pl.ds(r0, CH)], sA)
        pltpu.async_copy(bufB, GB_hbm.at[pl.ds(r0, CH)], sB)

    def wait_write(bufA, bufB, sA, sB):
        pltpu.make_async_copy(bufA, GA_hbm.at[pl.ds(0, CH)], sA).wait()
        pltpu.make_async_copy(bufB, GB_hbm.at[pl.ds(0, CH)], sB).wait()

    start_gather(0, bufA0, bufB0, sgA0, sgB0)
    start_gather(1, bufA1, bufB1, sgA1, sgB1)

    def body(k, carry):
        j0 = 2 * k
        wait_gather(bufA0, bufB0, sgA0, sgB0)
        start_write(j0, bufA0, bufB0, swA0, swB0)
        wait_gather(bufA1, bufB1, sgA1, sgB1)
        start_write(j0 + 1, bufA1, bufB1, swA1, swB1)
        wait_write(bufA0, bufB0, swA0, swB0)
        start_gather(j0 + 2, bufA0, bufB0, sgA0, sgB0)
        wait_write(bufA1, bufB1, swA1, swB1)
        start_gather(j0 + 3, bufA1, bufB1, sgA1, sgB1)
        return carry

    lax.fori_loop(0, KJ // 2 - 1, body, 0)

    wait_gather(bufA0, bufB0, sgA0, sgB0)
    start_write(KJ - 2, bufA0, bufB0, swA0, swB0)
    wait_gather(bufA1, bufB1, sgA1, sgB1)
    start_write(KJ - 1, bufA1, bufB1, swA1, swB1)
    wait_write(bufA0, bufB0, swA0, swB0)
    wait_write(bufA1, bufB1, swA1, swB1)


def _phase1(A, B, srcT, dstT):
    mesh = plsc.VectorSubcoreMesh(core_axis_name="c", subcore_axis_name="s")
    f = functools.partial(
        pl.kernel,
        mesh=mesh,
        out_type=[
            jax.ShapeDtypeStruct((EP, D), _f32),
            jax.ShapeDtypeStruct((EP, D), _f32),
        ],
        scratch_types=[
            pltpu.VMEM((KJ, CH), jnp.int32),
            pltpu.VMEM((KJ, CH), jnp.int32),
            pltpu.VMEM((CH, D), _f32),
            pltpu.VMEM((CH, D), _f32),
            pltpu.VMEM((CH, D), _f32),
            pltpu.VMEM((CH, D), _f32),
        ] + [pltpu.SemaphoreType.DMA] * 8,
    )(_p1_body)
    return f(A, B, srcT, dstT)


def _p2_body(ef, ga, gb, beW, beB, Wc, crow, ceW1, ceB1,
             eo_o, eu_o, esum_o):
    i = pl.program_id(0)
    e = jax.nn.relu(jnp.dot(ef[...], beW[...], preferred_element_type=_f32)
                    + beB[...])
    h = jax.nn.relu(jnp.dot(e, Wc[...], preferred_element_type=_f32)
                    + ga[...] + gb[...] + crow[...])
    eu = jax.nn.relu(jnp.dot(h, ceW1[...], preferred_element_type=_f32)
                     + ceB1[...])
    eu_o[...] = eu
    eo_o[...] = eu + ef[...]
    part = jnp.sum(eu, axis=0, keepdims=True)

    @pl.when(i == 0)
    def _():
        esum_o[...] = part

    @pl.when(i > 0)
    def _():
        esum_o[...] = esum_o[...] + part


def _phase2(ef, GA, GB, beW, beB, Wc, crow_e, ceW1, ceB1):
    BLK = 3200
    grid = E // BLK
    full = pl.BlockSpec((D, D), lambda i: (0, 0))
    row = pl.BlockSpec((1, D), lambda i: (0, 0))
    blk = pl.BlockSpec((BLK, D), lambda i: (i, 0))
    return pl.pallas_call(
        _p2_body,
        grid=(grid,),
        in_specs=[blk, blk, blk, full, row, full, row, full, row],
        out_specs=[blk, blk, row],
        out_shape=[
            jax.ShapeDtypeStruct((E, D), _f32),
            jax.ShapeDtypeStruct((EP, D), _f32),
            jax.ShapeDtypeStruct((1, D), _f32),
        ],
        name="p2_edge_pipeline",
        compiler_params=pltpu.CompilerParams(
            dimension_semantics=("arbitrary",)),
    )(ef, GA, GB, beW, beB.reshape(1, D), Wc, crow_e, ceW1,
      ceB1.reshape(1, D))


def _p3_body(eu_hbm, dstT, sums_hbm, counts_hbm,
             dst_v, buf, ones1, zb1, acc_sh, cnt_sh):
    c = lax.axis_index("c")
    s = lax.axis_index("s")
    wid = s * 2 + c
    base = wid * KJ
    pltpu.sync_copy(dstT.at[pl.ds(base, KJ)], dst_v)

    z16 = jnp.zeros((16,), _f32)
    ones = jnp.full((16,), 1.0, _f32)

    def zero_row(i, carry):
        for t in range(8):
            buf[i, pl.ds(t * 16, 16)] = z16
        return carry

    lax.fori_loop(0, CH, zero_row, 0)

    def fill_ones(i, carry):
        ones1[pl.ds(i * 16, 16)] = ones
        return carry

    lax.fori_loop(0, CH // 16, fill_ones, 0)

    def zero_z(i, carry):
        zb1[pl.ds(i * 16, 16)] = z16
        return carry

    lax.fori_loop(0, 640 // 16, zero_z, 0)

    for k in range(5):
        pltpu.sync_copy(buf, acc_sh.at[pl.ds(s * 640 + k * CH, CH)])
    pltpu.sync_copy(zb1, cnt_sh.at[pl.ds(s * 640, 640)])

    plsc.subcore_barrier()

    def body(j, carry):
        r0 = (base + j) * CH
        pltpu.sync_copy(eu_hbm.at[pl.ds(r0, CH)], buf)
        pltpu.sync_copy(buf, acc_sh.at[dst_v.at[j]], add=True)
        pltpu.sync_copy(ones1, cnt_sh.at[dst_v.at[j]], add=True)
        return carry

    lax.fori_loop(0, KJ, body, 0)

    plsc.subcore_barrier()

    pltpu.sync_copy(cnt_sh.at[pl.ds(s * 640, 640)], zb1)
    pltpu.sync_copy(zb1, counts_hbm.at[c, pl.ds(s * 640, 640)])
    for k in range(5):
        r = s * 640 + k * CH
        pltpu.sync_copy(acc_sh.at[pl.ds(r, CH)], buf)
        pltpu.sync_copy(buf, sums_hbm.at[c, pl.ds(r, CH)])


def _phase3(eu_pad, dstT):
    mesh = plsc.VectorSubcoreMesh(core_axis_name="c", subcore_axis_name="s")
    f = functools.partial(
        pl.kernel,
        mesh=mesh,
        out_type=[
            jax.ShapeDtypeStruct((2, NPAD, D), _f32),
            jax.ShapeDtypeStruct((2, NPAD), _f32),
        ],
        scratch_types=[
            pltpu.VMEM((KJ, CH), jnp.int32),
            pltpu.VMEM((CH, D), _f32),
            pltpu.VMEM((CH,), _f32),
            pltpu.VMEM((640,), _f32),
            pltpu.VMEM_SHARED((NPAD, D), _f32),
            pltpu.VMEM_SHARED((NPAD,), _f32),
        ],
    )(_p3_body)
    return f(eu_pad, dstT)


def _p4_body(p, cntb, n, nf, crow_n, Wna, Wnb, cnW1, cnB1,
             st, esum, Wsa, Wsb, Wsc, csB0, csW1, csB1, sf,
             no_o, so_o, acc):
    i = pl.program_id(0)
    ng = pl.num_programs(0)
    sums = p[0] + p[1]
    ve = sums / jnp.maximum(cntb[...], 1.0)
    pre = (jnp.dot(n[...], Wna[...], preferred_element_type=_f32)
           + jnp.dot(ve, Wnb[...], preferred_element_type=_f32)
           + crow_n[...])
    nu = jax.nn.relu(jnp.dot(jax.nn.relu(pre), cnW1[...],
                             preferred_element_type=_f32) + cnB1[...])
    no_o[...] = nu + nf[...]

    rid = i * n.shape[0] + lax.broadcasted_iota(jnp.int32, nu.shape, 0)
    num = jnp.where(rid < N, nu, 0.0)
    part = jnp.sum(num, axis=0, keepdims=True)
    tot = jnp.where(i == 0, part, acc[...] + part)
    acc[...] = tot

    @pl.when(i == ng - 1)
    def _():
        u_e = esum[...] * (1.0 / E)
        u_v = tot * (1.0 / N)
        t = jax.nn.relu(
            jnp.dot(st[...], Wsa[...], preferred_element_type=_f32)
            + jnp.dot(u_e, Wsb[...], preferred_element_type=_f32)
            + jnp.dot(u_v, Wsc[...], preferred_element_type=_f32)
            + csB0[...])
        su = jax.nn.relu(jnp.dot(t, csW1[...],
                                 preferred_element_type=_f32) + csB1[...])
        so_o[...] = su + sf[...]


def _phase4(parts, cntb, n_pad, nf_pad, crow_n, Wna, Wnb, cnW1, cnB1,
            st, esum, Wsa, Wsb, Wsc, csB0, csW1, csB1, sf):
    BLK = 1024
    grid = NPAD // BLK
    full = pl.BlockSpec((D, D), lambda i: (0, 0))
    row = pl.BlockSpec((1, D), lambda i: (0, 0))
    blk = pl.BlockSpec((BLK, D), lambda i: (i, 0))
    pblk = pl.BlockSpec((2, BLK, D), lambda i: (0, i, 0))
    return pl.pallas_call(
        _p4_body,
        grid=(grid,),
        in_specs=[pblk, blk, blk, blk, row, full, full, full, row,
                  row, row, full, full, full, row, full, row, row],
        out_specs=[blk, row],
        out_shape=[
            jax.ShapeDtypeStruct((NPAD, D), _f32),
            jax.ShapeDtypeStruct((1, D), _f32),
        ],
        scratch_shapes=[pltpu.VMEM((1, D), _f32)],
        compiler_params=pltpu.CompilerParams(
            dimension_semantics=("arbitrary",)),
    )(parts, cntb, n_pad, nf_pad, crow_n, Wna, Wnb, cnW1,
      cnB1.reshape(1, D), st, esum, Wsa, Wsb, Wsc, csB0.reshape(1, D),
      csW1, csB1.reshape(1, D), sf)


def kernel(edge_feat, node_feat, state_feat, beW, beB, bnW, bnB, bsW, bsB,
           ceW0, ceB0, ceW1, ceB1, cnW0, cnB0, cnW1, cnB1,
           csW0, csB0, csW1, csB1, edge_index):
    Wa = ceW0[0 * D:1 * D]
    Wb = ceW0[1 * D:2 * D]
    Wc = ceW0[2 * D:3 * D]
    Wd = ceW0[3 * D:4 * D]
    Wna = cnW0[0 * D:1 * D]
    Wnb = cnW0[1 * D:2 * D]
    Wnc = cnW0[2 * D:3 * D]
    Wsa = csW0[0 * D:1 * D]
    Wsb = csW0[1 * D:2 * D]
    Wsc = csW0[2 * D:3 * D]

    nf_pad = jnp.pad(node_feat, ((0, NPAD - N), (0, 0)))
    src = edge_index[0]
    dst = edge_index[1]
    srcT = jnp.pad(src, (0, EP - E)).reshape(EP // CH, CH)
    dstT = jnp.pad(dst, (0, EP - E),
                   constant_values=TRASH).reshape(EP // CH, CH)
    n_pad, A, B, st, crow_e, crow_n = _phase0(
        nf_pad, state_feat, bnW, bnB, bsW, bsB, Wa, Wb, Wd, ceB0, Wnc, cnB0)

    GA, GB = _phase1(A, B, srcT, dstT)

    e_out, eu_pad, esum = _phase2(edge_feat, GA, GB, beW, beB, Wc, crow_e,
                                  ceW1, ceB1)

    sums_parts, counts_parts = _phase3(eu_pad, dstT)

    cnt = (counts_parts[0] + counts_parts[1]).reshape(NPAD, 1)
    cntb = jnp.broadcast_to(cnt, (NPAD, D))

    n_out_pad, s_out = _phase4(
        sums_parts, cntb, n_pad, nf_pad, crow_n, Wna, Wnb, cnW1, cnB1,
        st, esum, Wsa, Wsb, Wsc, csB0, csW1, csB1, state_feat)

    return (e_out, n_out_pad[:N], s_out)


# speedup vs baseline: 3.0969x; 1.0199x over previous
import functools

import jax
import jax.numpy as jnp
from jax import lax
from jax.experimental import pallas as pl
from jax.experimental.pallas import tpu as pltpu
from jax.experimental.pallas import tpu_sc as plsc

N = 10000
E = 320000
D = 128
NPAD = 10240
NW = 32
CH = 128
KJ = 80
EW = CH * KJ
EP = NW * EW
TRASH = NPAD - 1

_f32 = jnp.float32


def _p0_body(nf, sf, bnW, bnB, bsW, bsB, Wa, Wb, Wd, ceB0, Wnc, cnB0,
             n_o, A_o, B_o, st_o, crow_e_o, crow_n_o):
    i = pl.program_id(0)
    n = jax.nn.relu(jnp.dot(nf[...], bnW[...], preferred_element_type=_f32)
                    + bnB[...])
    n_o[...] = n
    a = jnp.dot(n, Wa[...], preferred_element_type=_f32)
    b = jnp.dot(n, Wb[...], preferred_element_type=_f32)
    A_o[...] = jnp.broadcast_to(a[None], A_o.shape)
    B_o[...] = jnp.broadcast_to(b[None], B_o.shape)

    @pl.when(i == 0)
    def _():
        st = jax.nn.relu(jnp.dot(sf[...], bsW[...],
                                 preferred_element_type=_f32) + bsB[...])
        st_o[...] = st
        crow_e_o[...] = jnp.dot(st, Wd[...],
                                preferred_element_type=_f32) + ceB0[...]
        crow_n_o[...] = jnp.dot(st, Wnc[...],
                                preferred_element_type=_f32) + cnB0[...]


def _phase0(nf_pad, sf, bnW, bnB, bsW, bsB, Wa, Wb, Wd, ceB0, Wnc, cnB0):
    BLK = 1024
    grid = NPAD // BLK
    full = pl.BlockSpec((D, D), lambda i: (0, 0))
    row = pl.BlockSpec((1, D), lambda i: (0, 0))
    blk = pl.BlockSpec((BLK, D), lambda i: (i, 0))
    blk2 = pl.BlockSpec((2, BLK, D), lambda i: (0, i, 0))
    return pl.pallas_call(
        _p0_body,
        grid=(grid,),
        in_specs=[blk, row, full, row, full, row, full, full, full, row,
                  full, row],
        out_specs=[blk, blk2, blk2, row, row, row],
        out_shape=[
            jax.ShapeDtypeStruct((NPAD, D), _f32),
            jax.ShapeDtypeStruct((2, NPAD, D), _f32),
            jax.ShapeDtypeStruct((2, NPAD, D), _f32),
            jax.ShapeDtypeStruct((1, D), _f32),
            jax.ShapeDtypeStruct((1, D), _f32),
            jax.ShapeDtypeStruct((1, D), _f32),
        ],
        compiler_params=pltpu.CompilerParams(
            dimension_semantics=("arbitrary",)),
    )(nf_pad, sf, bnW, bnB.reshape(1, D), bsW, bsB.reshape(1, D), Wa, Wb,
      Wd, ceB0.reshape(1, D), Wnc, cnB0.reshape(1, D))


def _p1_body(A_hbm, B_hbm, srcT, dstT, GA_hbm, GB_hbm,
             src_v, dst_v, bufA0, bufB0, bufA1, bufB1,
             sgA0, sgB0, sgA1, sgB1, swA0, swB0, swA1, swB1):
    c = lax.axis_index("c")
    s = lax.axis_index("s")
    wid = s * 2 + c
    base = wid * KJ
    pltpu.sync_copy(srcT.at[pl.ds(base, KJ)], src_v)
    pltpu.sync_copy(dstT.at[pl.ds(base, KJ)], dst_v)

    def start_gather(j, bufA, bufB, sA, sB):
        pltpu.async_copy(A_hbm.at[c].at[src_v.at[j]], bufA, sA)
        pltpu.async_copy(B_hbm.at[c].at[dst_v.at[j]], bufB, sB)

    def wait_gather(bufA, bufB, sA, sB):
        pltpu.make_async_copy(A_hbm.at[c].at[src_v.at[0]], bufA, sA).wait()
        pltpu.make_async_copy(B_hbm.at[c].at[dst_v.at[0]], bufB, sB).wait()

    def start_write(j, bufA, bufB, sA, sB):
        r0 = (base + j) * CH
        pltpu.async_copy(bufA, GA_hbm.at[pl.ds(r0, CH)], sA)
        pltpu.async_copy(bufB, GB_hbm.at[pl.ds(r0, CH)], sB)

    def wait_write(bufA, bufB, sA, sB):
        pltpu.make_async_copy(bufA, GA_hbm.at[pl.ds(0, CH)], sA).wait()
        pltpu.make_async_copy(bufB, GB_hbm.at[pl.ds(0, CH)], sB).wait()

    start_gather(0, bufA0, bufB0, sgA0, sgB0)
    start_gather(1, bufA1, bufB1, sgA1, sgB1)

    def body(k, carry):
        j0 = 2 * k
        wait_gather(bufA0, bufB0, sgA0, sgB0)
        start_write(j0, bufA0, bufB0, swA0, swB0)
        wait_gather(bufA1, bufB1, sgA1, sgB1)
        start_write(j0 + 1, bufA1, bufB1, swA1, swB1)
        wait_write(bufA0, bufB0, swA0, swB0)
        start_gather(j0 + 2, bufA0, bufB0, sgA0, sgB0)
        wait_write(bufA1, bufB1, swA1, swB1)
        start_gather(j0 + 3, bufA1, bufB1, sgA1, sgB1)
        return carry

    lax.fori_loop(0, KJ // 2 - 1, body, 0)

    wait_gather(bufA0, bufB0, sgA0, sgB0)
    start_write(KJ - 2, bufA0, bufB0, swA0, swB0)
    wait_gather(bufA1, bufB1, sgA1, sgB1)
    start_write(KJ - 1, bufA1, bufB1, swA1, swB1)
    wait_write(bufA0, bufB0, swA0, swB0)
    wait_write(bufA1, bufB1, swA1, swB1)


def _phase1(A, B, srcT, dstT):
    mesh = plsc.VectorSubcoreMesh(core_axis_name="c", subcore_axis_name="s")
    f = functools.partial(
        pl.kernel,
        mesh=mesh,
        out_type=[
            jax.ShapeDtypeStruct((EP, D), _f32),
            jax.ShapeDtypeStruct((EP, D), _f32),
        ],
        scratch_types=[
            pltpu.VMEM((KJ, CH), jnp.int32),
            pltpu.VMEM((KJ, CH), jnp.int32),
            pltpu.VMEM((CH, D), _f32),
            pltpu.VMEM((CH, D), _f32),
            pltpu.VMEM((CH, D), _f32),
            pltpu.VMEM((CH, D), _f32),
        ] + [pltpu.SemaphoreType.DMA] * 8,
    )(_p1_body)
    return f(A, B, srcT, dstT)


def _p2_body(ef, ga, gb, beW, beB, Wc, crow, ceW1, ceB1,
             eo_o, eu_o, esum_o):
    i = pl.program_id(0)
    e = jax.nn.relu(jnp.dot(ef[...], beW[...], preferred_element_type=_f32)
                    + beB[...])
    h = jax.nn.relu(jnp.dot(e, Wc[...], preferred_element_type=_f32)
                    + ga[...] + gb[...] + crow[...])
    eu = jax.nn.relu(jnp.dot(h, ceW1[...], preferred_element_type=_f32)
                     + ceB1[...])
    eu_o[...] = eu
    eo_o[...] = eu + ef[...]
    part = jnp.sum(eu, axis=0, keepdims=True)

    @pl.when(i == 0)
    def _():
        esum_o[...] = part

    @pl.when(i > 0)
    def _():
        esum_o[...] = esum_o[...] + part


def _phase2(ef, GA, GB, beW, beB, Wc, crow_e, ceW1, ceB1):
    BLK = 3200
    grid = E // BLK
    full = pl.BlockSpec((D, D), lambda i: (0, 0))
    row = pl.BlockSpec((1, D), lambda i: (0, 0))
    blk = pl.BlockSpec((BLK, D), lambda i: (i, 0))
    return pl.pallas_call(
        _p2_body,
        grid=(grid,),
        in_specs=[blk, blk, blk, full, row, full, row, full, row],
        out_specs=[blk, blk, row],
        out_shape=[
            jax.ShapeDtypeStruct((E, D), _f32),
            jax.ShapeDtypeStruct((EP, D), _f32),
            jax.ShapeDtypeStruct((1, D), _f32),
        ],
        name="p2_edge_pipeline",
        compiler_params=pltpu.CompilerParams(
            dimension_semantics=("arbitrary",)),
    )(ef, GA, GB, beW, beB.reshape(1, D), Wc, crow_e, ceW1,
      ceB1.reshape(1, D))


def _p3_body(eu_hbm, dstT, sums_hbm, counts_hbm,
             dst_v, buf, ones1, zb1, seg, out1, acc_sh, cnt_sh):
    c = lax.axis_index("c")
    s = lax.axis_index("s")
    wid = s * 2 + c
    base = wid * KJ
    pltpu.sync_copy(dstT.at[pl.ds(base, KJ)], dst_v)

    z16 = jnp.zeros((16,), _f32)
    ones = jnp.full((16,), 1.0, _f32)

    def zero_row(i, carry):
        for t in range(8):
            buf[i, pl.ds(t * 16, 16)] = z16
        return carry

    lax.fori_loop(0, CH, zero_row, 0)

    def fill_ones(i, carry):
        ones1[pl.ds(i * 16, 16)] = ones
        return carry

    lax.fori_loop(0, CH // 16, fill_ones, 0)

    def zero_z(i, carry):
        zb1[pl.ds(i * 16, 16)] = z16
        return carry

    lax.fori_loop(0, 640 // 16, zero_z, 0)

    for k in range(5):
        pltpu.sync_copy(buf, acc_sh.at[pl.ds(s * 640 + k * CH, CH)])
    for k in range(16):
        pltpu.sync_copy(zb1, cnt_sh.at[pl.ds(s * NPAD + k * 640, 640)])

    plsc.subcore_barrier()

    def body(j, carry):
        r0 = (base + j) * CH
        pltpu.sync_copy(eu_hbm.at[pl.ds(r0, CH)], buf)
        pltpu.sync_copy(buf, acc_sh.at[dst_v.at[j]], add=True)
        pltpu.sync_copy(ones1, cnt_sh.at[pl.ds(s * NPAD, NPAD)]
                        .at[dst_v.at[j]], add=True)
        return carry

    lax.fori_loop(0, KJ, body, 0)

    plsc.subcore_barrier()

    def zero_o(i, carry):
        out1[pl.ds(i * 16, 16)] = z16
        return carry

    lax.fori_loop(0, 640 // 16, zero_o, 0)

    for k in range(16):
        pltpu.sync_copy(cnt_sh.at[pl.ds(k * NPAD + s * 640, 640)], seg)

        def addseg(i, carry):
            out1[pl.ds(i * 16, 16)] = (out1[pl.ds(i * 16, 16)]
                                       + seg[pl.ds(i * 16, 16)])
            return carry

        lax.fori_loop(0, 640 // 16, addseg, 0)

    pltpu.sync_copy(out1, counts_hbm.at[c, pl.ds(s * 640, 640)])
    for k in range(5):
        r = s * 640 + k * CH
        pltpu.sync_copy(acc_sh.at[pl.ds(r, CH)], buf)
        pltpu.sync_copy(buf, sums_hbm.at[c, pl.ds(r, CH)])


def _phase3(eu_pad, dstT):
    mesh = plsc.VectorSubcoreMesh(core_axis_name="c", subcore_axis_name="s")
    f = functools.partial(
        pl.kernel,
        mesh=mesh,
        out_type=[
            jax.ShapeDtypeStruct((2, NPAD, D), _f32),
            jax.ShapeDtypeStruct((2, NPAD), _f32),
        ],
        scratch_types=[
            pltpu.VMEM((KJ, CH), jnp.int32),
            pltpu.VMEM((CH, D), _f32),
            pltpu.VMEM((CH,), _f32),
            pltpu.VMEM((640,), _f32),
            pltpu.VMEM((640,), _f32),
            pltpu.VMEM((640,), _f32),
            pltpu.VMEM_SHARED((NPAD, D), _f32),
            pltpu.VMEM_SHARED((16 * NPAD,), _f32),
        ],
    )(_p3_body)
    return f(eu_pad, dstT)


def _p4_body(p, cntb, n, nf, crow_n, Wna, Wnb, cnW1, cnB1,
             st, esum, Wsa, Wsb, Wsc, csB0, csW1, csB1, sf,
             no_o, so_o, acc):
    i = pl.program_id(0)
    ng = pl.num_programs(0)
    sums = p[0] + p[1]
    ve = sums / jnp.maximum(cntb[...], 1.0)
    pre = (jnp.dot(n[...], Wna[...], preferred_element_type=_f32)
           + jnp.dot(ve, Wnb[...], preferred_element_type=_f32)
           + crow_n[...])
    nu = jax.nn.relu(jnp.dot(jax.nn.relu(pre), cnW1[...],
                             preferred_element_type=_f32) + cnB1[...])
    no_o[...] = nu + nf[...]

    rid = i * n.shape[0] + lax.broadcasted_iota(jnp.int32, nu.shape, 0)
    num = jnp.where(rid < N, nu, 0.0)
    part = jnp.sum(num, axis=0, keepdims=True)
    tot = jnp.where(i == 0, part, acc[...] + part)
    acc[...] = tot

    @pl.when(i == ng - 1)
    def _():
        u_e = esum[...] * (1.0 / E)
        u_v = tot * (1.0 / N)
        t = jax.nn.relu(
            jnp.dot(st[...], Wsa[...], preferred_element_type=_f32)
            + jnp.dot(u_e, Wsb[...], preferred_element_type=_f32)
            + jnp.dot(u_v, Wsc[...], preferred_element_type=_f32)
            + csB0[...])
        su = jax.nn.relu(jnp.dot(t, csW1[...],
                                 preferred_element_type=_f32) + csB1[...])
        so_o[...] = su + sf[...]


def _phase4(parts, cntb, n_pad, nf_pad, crow_n, Wna, Wnb, cnW1, cnB1,
            st, esum, Wsa, Wsb, Wsc, csB0, csW1, csB1, sf):
    BLK = 1024
    grid = NPAD // BLK
    full = pl.BlockSpec((D, D), lambda i: (0, 0))
    row = pl.BlockSpec((1, D), lambda i: (0, 0))
    blk = pl.BlockSpec((BLK, D), lambda i: (i, 0))
    pblk = pl.BlockSpec((2, BLK, D), lambda i: (0, i, 0))
    return pl.pallas_call(
        _p4_body,
        grid=(grid,),
        in_specs=[pblk, blk, blk, blk, row, full, full, full, row,
                  row, row, full, full, full, row, full, row, row],
        out_specs=[blk, row],
        out_shape=[
            jax.ShapeDtypeStruct((NPAD, D), _f32),
            jax.ShapeDtypeStruct((1, D), _f32),
        ],
        scratch_shapes=[pltpu.VMEM((1, D), _f32)],
        compiler_params=pltpu.CompilerParams(
            dimension_semantics=("arbitrary",)),
    )(parts, cntb, n_pad, nf_pad, crow_n, Wna, Wnb, cnW1,
      cnB1.reshape(1, D), st, esum, Wsa, Wsb, Wsc, csB0.reshape(1, D),
      csW1, csB1.reshape(1, D), sf)


def kernel(edge_feat, node_feat, state_feat, beW, beB, bnW, bnB, bsW, bsB,
           ceW0, ceB0, ceW1, ceB1, cnW0, cnB0, cnW1, cnB1,
           csW0, csB0, csW1, csB1, edge_index):
    Wa = ceW0[0 * D:1 * D]
    Wb = ceW0[1 * D:2 * D]
    Wc = ceW0[2 * D:3 * D]
    Wd = ceW0[3 * D:4 * D]
    Wna = cnW0[0 * D:1 * D]
    Wnb = cnW0[1 * D:2 * D]
    Wnc = cnW0[2 * D:3 * D]
    Wsa = csW0[0 * D:1 * D]
    Wsb = csW0[1 * D:2 * D]
    Wsc = csW0[2 * D:3 * D]

    nf_pad = jnp.pad(node_feat, ((0, NPAD - N), (0, 0)))
    src = edge_index[0]
    dst = edge_index[1]
    srcT = jnp.pad(src, (0, EP - E)).reshape(EP // CH, CH)
    dstT = jnp.pad(dst, (0, EP - E),
                   constant_values=TRASH).reshape(EP // CH, CH)
    n_pad, A, B, st, crow_e, crow_n = _phase0(
        nf_pad, state_feat, bnW, bnB, bsW, bsB, Wa, Wb, Wd, ceB0, Wnc, cnB0)

    GA, GB = _phase1(A, B, srcT, dstT)

    e_out, eu_pad, esum = _phase2(edge_feat, GA, GB, beW, beB, Wc, crow_e,
                                  ceW1, ceB1)

    sums_parts, counts_parts = _phase3(eu_pad, dstT)

    cnt = (counts_parts[0] + counts_parts[1]).reshape(NPAD, 1)
    cntb = jnp.broadcast_to(cnt, (NPAD, D))

    n_out_pad, s_out = _phase4(
        sums_parts, cntb, n_pad, nf_pad, crow_n, Wna, Wnb, cnW1, cnB1,
        st, esum, Wsa, Wsb, Wsc, csB0, csW1, csB1, state_feat)

    return (e_out, n_out_pad[:N], s_out)
